# baseline (speedup 1.0000x reference)
import functools

import jax
import jax.numpy as jnp
from jax import lax
from jax.experimental import pallas as pl
from jax.experimental.pallas import tpu as pltpu

B = 2048
IN = 256
OUT = 128
LANE = 128
NCHUNK = B // LANE
OGRP = 8


def _xor_perm1(a, j):
    iota = lax.broadcasted_iota(jnp.int32, a.shape, 1)
    bit = (iota & j) != 0
    up = jnp.roll(a, j, axis=1)
    dn = jnp.roll(a, -j, axis=1)
    return jnp.where(bit, up, dn)


def _loss_body(xT_ref, tT_ref, wT_ref, out_ref, sT_ref):
    f32 = jnp.float32
    ones_col = jnp.ones((LANE, 1), f32)

    dacc = xT_ref[:, 0:LANE]
    for c in range(1, NCHUNK):
        dacc = dacc + xT_ref[:, c * LANE:(c + 1) * LANE]
    denomT = lax.dot_general(ones_col, dacc, (((0,), (1,)), ((), ())),
                             preferred_element_type=f32)

    def _compute_accs(s):
        o0 = (s // 2) * OGRP
        r0 = (s % 2) * (IN // 2)
        accs = [None] * OGRP
        for c in range(NCHUNK):
            cs = slice(c * LANE, (c + 1) * LANE)
            xc = xT_ref[pl.ds(r0, IN // 2), cs]
            t8 = tT_ref[pl.ds(o0, OGRP), cs]
            for r in range(OGRP):
                trow = lax.slice(t8, (r, 0), (r + 1, LANE))
                m = jnp.minimum(xc, trow)
                accs[r] = m if c == 0 else accs[r] + m
        return accs

    def _reduce_store(s, accs):
        srows = [lax.dot_general(ones_col, accs[r], (((0,), (1,)), ((), ())),
                                 preferred_element_type=f32)
                 for r in range(OGRP)]
        o0 = (s // 2) * OGRP
        r0 = (s % 2) * (IN // 2)
        sT_ref[pl.ds(o0, OGRP), pl.ds(r0, IN // 2)] = jnp.concatenate(srows, 0)

    def sbody(s, accs):
        new_accs = _compute_accs(s)
        _reduce_store(s - 1, accs)
        return tuple(new_accs)

    last = lax.fori_loop(1, 2 * (OUT // OGRP), sbody, tuple(_compute_accs(0)))
    _reduce_store(2 * (OUT // OGRP) - 1, last)

    sT = sT_ref[...]
    scoreT = jnp.where(denomT == 0.0, jnp.float32(1.0), sT / denomT)

    iota1 = lax.broadcasted_iota(jnp.int32, (OUT, IN), 1)
    key = scoreT
    pay = wT_ref[...]
    sw = wT_ref[...]
    for k in [2, 4, 8, 16, 32, 64, 128, 256]:
        j = k // 2
        while j >= 1:
            kp = _xor_perm1(key, j)
            pp = _xor_perm1(pay, j)
            swp = _xor_perm1(sw, j)
            is_lower = (iota1 & j) == 0
            before = (key > kp) | (is_lower & (key == kp))
            pbits = iota1 & (k + j)
            flip = (pbits == k) | (pbits == j)
            keep = before != flip
            key = jnp.where(keep, key, kp)
            pay = jnp.where(keep, pay, pp)
            d = (iota1 & k) == 0
            hi = jnp.maximum(sw, swp)
            lo = jnp.minimum(sw, swp)
            sw = jnp.where(is_lower == d, hi, lo)
            j //= 2
    target_w = pay
    sorted_w = sw

    diff = sorted_w - target_w
    sq = diff * diff
    total = jnp.sum(jnp.sum(sq, axis=0, keepdims=True), axis=1, keepdims=True)
    out_ref[...] = total / jnp.float32(IN * OUT)


@functools.partial(jax.jit, static_argnames=("interpret",))
def _run(x, t, w, interpret=False):
    xT = x.T
    tT = t.T
    wT = w.T
    out = pl.pallas_call(
        _loss_body,
        out_shape=jax.ShapeDtypeStruct((1, 1), jnp.float32),
        scratch_shapes=[pltpu.VMEM((OUT, IN), jnp.float32)],
        interpret=interpret,
    )(xT, tT, wT)
    return out[0, 0]


def kernel(x, y, t, w):
    del y
    return _run(x, t, w)
